# baseline (speedup 1.0000x reference)
import functools

import jax
import jax.numpy as jnp
from jax import lax
from jax.experimental import pallas as pl
from jax.experimental.pallas import tpu as pltpu
from jax.experimental.pallas import tpu_sc as plsc

N = 10000
E = 320000
D = 128
H = D // 2

NC = 2
NS = 16
K = 64
EPW = 19968
CHUNKS = EPW // K
PAIRS = CHUNKS // 2
TAIL_BASE = NS * EPW
NP_ = 10240
ROWS_PT = NP_ // NS


def _proj_body(h_ref, wa, ba, wb, bb, wd, bd_, we, be, ah_ref, bdt_ref, et_ref):
    hb = h_ref[...]
    ah_ref[...] = jnp.dot(hb, wa[...], preferred_element_type=jnp.float32) + ba[...]
    yb = jnp.dot(hb, wb[...], preferred_element_type=jnp.float32) + bb[...]
    yd = jnp.dot(hb, wd[...], preferred_element_type=jnp.float32) + bd_[...]
    et_ref[...] = jnp.dot(hb, we[...], preferred_element_type=jnp.float32) + be[...]
    for c in range(NC):
        bdt_ref[c, :, 0:H] = yb[:, c * H:(c + 1) * H]
        bdt_ref[c, :, H:D] = yd[:, c * H:(c + 1) * H]


def _edge_body(bdt, et, src_hbm, dst_hbm, out,
               ibd0, ibd1, dst0, dst1,
               bd_a, bd_b, ev_a, ev_b, ms_v,
               acc_sh, isem0, isem1, gsem0, gsem1, esem0, esem1):
    c = lax.axis_index("c")
    s = lax.axis_index("s")
    cN = c * N
    cH = c * H
    ibd = (ibd0, ibd1)
    dstg = (dst0, dst1)
    bdv = (bd_a, bd_b)
    evv = (ev_a, ev_b)
    isem = (isem0, isem1)
    gsem = (gsem0, gsem1)
    esem = (esem0, esem1)

    def issue_idx(base, slot):
        pltpu.async_copy(src_hbm.at[pl.ds(base, K)], ibd[slot], isem[slot])
        pltpu.async_copy(dst_hbm.at[pl.ds(base, K)], dstg[slot], isem[slot])

    def wait_idx(slot):
        pltpu.make_async_copy(src_hbm.at[pl.ds(0, K)], ibd[slot],
                              isem[slot]).wait()
        pltpu.make_async_copy(dst_hbm.at[pl.ds(0, K)], dstg[slot],
                              isem[slot]).wait()

    def issue_gather(slot):
        for f in range(K // 16):
            sl = pl.ds(f * 16, 16)
            ibd[slot][sl] = ibd[slot][sl] + cN
        pltpu.async_copy(bdt.at[ibd[slot]], bdv[slot], gsem[slot])
        pltpu.async_copy(et.at[dstg[slot]], evv[slot], esem[slot])

    def wait_gather(slot):
        pltpu.make_async_copy(bdt.at[ibd[slot]], bdv[slot], gsem[slot]).wait()
        pltpu.make_async_copy(et.at[dstg[slot]], evv[slot], esem[slot]).wait()

    def compute(slot):
        bd_v = bdv[slot]
        e_v = evv[slot]

        @plsc.parallel_loop(0, K, unroll=8)
        def edge(i):
            for f in range(H // 16):
                lo = pl.ds(f * 16, 16)
                hi = pl.ds(H + f * 16, 16)
                b = bd_v[i, lo]
                d = bd_v[i, hi]
                ev = e_v[i, pl.ds(cH + f * 16, 16)]
                sg = 1.0 / (1.0 + jnp.exp(-(d + ev)))
                ms_v[i, lo] = sg * b
                ms_v[i, hi] = sg

    def scatter(slot):
        pltpu.sync_copy(ms_v, acc_sh.at[dstg[slot]], add=True)

    issue_idx(s * EPW, 0)
    issue_idx(s * EPW + K, 1)

    def zrow(i, _):
        for f in range(D // 16):
            ms_v[i, pl.ds(f * 16, 16)] = jnp.zeros((16,), jnp.float32)
        return 0
    lax.fori_loop(0, K, zrow, 0)
    for r in range(ROWS_PT // K):
        pltpu.sync_copy(ms_v, acc_sh.at[pl.ds(s * ROWS_PT + r * K, K)])
    plsc.subcore_barrier()

    wait_idx(0)
    issue_gather(0)

    def pair(p, _):
        for off in (0, 1):
            slot = off
            nslot = 1 - off
            t = 2 * p + off
            wait_gather(slot)
            if off == 0:
                wait_idx(nslot)
                issue_gather(nslot)
            else:
                @pl.when(p < PAIRS - 1)
                def _():
                    wait_idx(nslot)
                    issue_gather(nslot)

            compute(slot)
            scatter(slot)

            @pl.when(p < PAIRS - 1)
            def _():
                issue_idx(s * EPW + (t + 2) * K, slot)
        return 0
    lax.fori_loop(0, PAIRS, pair, 0)

    @pl.when(s < (E - TAIL_BASE) // K)
    def _():
        issue_idx(TAIL_BASE + s * K, 0)
        wait_idx(0)
        issue_gather(0)
        wait_gather(0)
        compute(0)
        scatter(0)

    plsc.subcore_barrier()
    pltpu.sync_copy(acc_sh.at[pl.ds(s * ROWS_PT, ROWS_PT)],
                    out.at[c, pl.ds(s * ROWS_PT, ROWS_PT)])


def _bn_body(ah_ref, acc_ref, g_ref, b_ref, hin_ref, out_ref):
    ah = ah_ref[...]
    hp_lo = ah[:, 0:H] + acc_ref[0, 0:N, 0:H] / (acc_ref[0, 0:N, H:D] + 1e-6)
    hp_hi = ah[:, H:D] + acc_ref[1, 0:N, 0:H] / (acc_ref[1, 0:N, H:D] + 1e-6)
    hp = jnp.concatenate([hp_lo, hp_hi], axis=1)
    mean = jnp.mean(hp, axis=0, keepdims=True)
    var = jnp.mean((hp - mean) ** 2, axis=0, keepdims=True)
    y = (hp - mean) * (g_ref[...] * lax.rsqrt(var + 1e-5)) + b_ref[...]
    out_ref[...] = hin_ref[...] + jnp.maximum(y, 0.0)


@jax.jit
def kernel(h, edge_index, e, WA, bA, WB, bB, WD, bD, WE, bE, gamma, beta):
    f32 = jnp.float32
    ah, bdt, et = pl.pallas_call(
        _proj_body,
        out_shape=(
            jax.ShapeDtypeStruct((N, D), f32),
            jax.ShapeDtypeStruct((NC, N, D), f32),
            jax.ShapeDtypeStruct((N, D), f32),
        ),
    )(h, WA, bA.reshape(1, D), WB, bB.reshape(1, D),
      WD, bD.reshape(1, D), WE, bE.reshape(1, D))

    mesh = plsc.VectorSubcoreMesh(core_axis_name="c", subcore_axis_name="s")
    acc = pl.kernel(
        _edge_body,
        out_type=jax.ShapeDtypeStruct((NC, NP_, D), f32),
        mesh=mesh,
        scratch_types=[
            pltpu.VMEM((K,), jnp.int32),
            pltpu.VMEM((K,), jnp.int32),
            pltpu.VMEM((K,), jnp.int32),
            pltpu.VMEM((K,), jnp.int32),
            pltpu.VMEM((K, D), f32),
            pltpu.VMEM((K, D), f32),
            pltpu.VMEM((K, D), f32),
            pltpu.VMEM((K, D), f32),
            pltpu.VMEM((K, D), f32),
            pltpu.VMEM_SHARED((NP_, D), f32),
            pltpu.SemaphoreType.DMA,
            pltpu.SemaphoreType.DMA,
            pltpu.SemaphoreType.DMA,
            pltpu.SemaphoreType.DMA,
            pltpu.SemaphoreType.DMA,
            pltpu.SemaphoreType.DMA,
        ],
    )(bdt.reshape(NC * N, D), et, edge_index[0], edge_index[1])

    h_new = pl.pallas_call(
        _bn_body,
        out_shape=jax.ShapeDtypeStruct((N, D), f32),
    )(ah, acc, gamma.reshape(1, D), beta.reshape(1, D), h)
    return (h_new, e)


# speedup vs baseline: 1.0473x; 1.0473x over previous
import functools

import jax
import jax.numpy as jnp
from jax import lax
from jax.experimental import pallas as pl
from jax.experimental.pallas import tpu as pltpu
from jax.experimental.pallas import tpu_sc as plsc

N = 10000
E = 320000
D = 128
H = D // 2

NC = 2
NS = 16
K = 64
EPW = 19968
CHUNKS = EPW // K
PAIRS = CHUNKS // 2
TAIL_BASE = NS * EPW
NP_ = 10112
ROWS_PT = NP_ // NS


def _proj_body(h_ref, wa, ba, wb, bb, wd, bd_, we, be, ah_ref, bdt_ref, et_ref):
    hb = h_ref[...]
    ah_ref[...] = jnp.dot(hb, wa[...], preferred_element_type=jnp.float32) + ba[...]
    yb = jnp.dot(hb, wb[...], preferred_element_type=jnp.float32) + bb[...]
    yd = jnp.dot(hb, wd[...], preferred_element_type=jnp.float32) + bd_[...]
    et_ref[...] = jnp.dot(hb, we[...], preferred_element_type=jnp.float32) + be[...]
    for c in range(NC):
        bdt_ref[c, :, 0:H] = yb[:, c * H:(c + 1) * H]
        bdt_ref[c, :, H:D] = yd[:, c * H:(c + 1) * H]


def _edge_body(bdt, et, src_hbm, dst_hbm, out,
               ibd0, ibd1, dst0, dst1, sd0, sd1,
               bd_a, bd_b, ev_a, ev_b, ms_a, ms_b,
               acc_sh, isem0, isem1, gsem0, gsem1, esem0, esem1,
               ssem0, ssem1):
    c = lax.axis_index("c")
    s = lax.axis_index("s")
    cN = c * N
    cH = c * H
    ibd = (ibd0, ibd1)
    dstg = (dst0, dst1)
    sdst = (sd0, sd1)
    bdv = (bd_a, bd_b)
    evv = (ev_a, ev_b)
    msv = (ms_a, ms_b)
    isem = (isem0, isem1)
    gsem = (gsem0, gsem1)
    esem = (esem0, esem1)
    ssem = (ssem0, ssem1)

    def issue_idx(base, slot):
        pltpu.async_copy(src_hbm.at[pl.ds(base, K)], ibd[slot], isem[slot])
        pltpu.async_copy(dst_hbm.at[pl.ds(base, K)], dstg[slot], isem[slot])

    def wait_idx(slot):
        pltpu.make_async_copy(src_hbm.at[pl.ds(0, K)], ibd[slot],
                              isem[slot]).wait()
        pltpu.make_async_copy(dst_hbm.at[pl.ds(0, K)], dstg[slot],
                              isem[slot]).wait()

    def issue_gather(slot):
        for f in range(K // 16):
            sl = pl.ds(f * 16, 16)
            ibd[slot][sl] = ibd[slot][sl] + cN
        pltpu.async_copy(bdt.at[ibd[slot]], bdv[slot], gsem[slot])
        pltpu.async_copy(et.at[dstg[slot]], evv[slot], esem[slot])

    def wait_gather(slot):
        pltpu.make_async_copy(bdt.at[ibd[slot]], bdv[slot], gsem[slot]).wait()
        pltpu.make_async_copy(et.at[dstg[slot]], evv[slot], esem[slot]).wait()

    def compute(slot):
        bd_v = bdv[slot]
        e_v = evv[slot]
        ms_v = msv[slot]

        @plsc.parallel_loop(0, K, unroll=4)
        def edge(i):
            for f in range(H // 16):
                lo = pl.ds(f * 16, 16)
                hi = pl.ds(H + f * 16, 16)
                b = bd_v[i, lo]
                d = bd_v[i, hi]
                ev = e_v[i, pl.ds(cH + f * 16, 16)]
                sg = 1.0 / (1.0 + jnp.exp(-(d + ev)))
                ms_v[i, lo] = sg * b
                ms_v[i, hi] = sg
        for f in range(K // 16):
            sl = pl.ds(f * 16, 16)
            sdst[slot][sl] = dstg[slot][sl]

    def issue_scatter(slot):
        pltpu.async_copy(msv[slot], acc_sh.at[sdst[slot]], ssem[slot],
                         add=True)

    def wait_scatter(slot):
        pltpu.make_async_copy(msv[slot], acc_sh.at[sdst[slot]],
                              ssem[slot]).wait()

    def scatter_sync(slot):
        pltpu.sync_copy(msv[slot], acc_sh.at[sdst[slot]], add=True)

    issue_idx(s * EPW, 0)
    issue_idx(s * EPW + K, 1)

    def zrow(i, _):
        for f in range(D // 16):
            ms_a[i, pl.ds(f * 16, 16)] = jnp.zeros((16,), jnp.float32)
        return 0
    lax.fori_loop(0, K, zrow, 0)
    for r in range(ROWS_PT // K):
        pltpu.sync_copy(ms_a, acc_sh.at[pl.ds(s * ROWS_PT + r * K, K)])
    rem = ROWS_PT - (ROWS_PT // K) * K
    if rem:
        pltpu.sync_copy(ms_a.at[pl.ds(0, rem)],
                        acc_sh.at[pl.ds(s * ROWS_PT + ROWS_PT - rem, rem)])
    plsc.subcore_barrier()

    wait_idx(0)
    issue_gather(0)

    def pair(p, _):
        for off in (0, 1):
            slot = off
            nslot = 1 - off
            t = 2 * p + off
            wait_gather(slot)
            if off == 0:
                wait_idx(nslot)
                issue_gather(nslot)
            else:
                @pl.when(p < PAIRS - 1)
                def _():
                    wait_idx(nslot)
                    issue_gather(nslot)

            @pl.when(p >= 1)
            def _():
                wait_scatter(slot)

            compute(slot)
            issue_scatter(slot)

            @pl.when(p < PAIRS - 1)
            def _():
                issue_idx(s * EPW + (t + 2) * K, slot)
        return 0
    lax.fori_loop(0, PAIRS, pair, 0)
    wait_scatter(0)
    wait_scatter(1)

    @pl.when(s < (E - TAIL_BASE) // K)
    def _():
        issue_idx(TAIL_BASE + s * K, 0)
        wait_idx(0)
        issue_gather(0)
        wait_gather(0)
        compute(0)
        scatter_sync(0)

    plsc.subcore_barrier()
    pltpu.sync_copy(acc_sh.at[pl.ds(s * ROWS_PT, ROWS_PT)],
                    out.at[c, pl.ds(s * ROWS_PT, ROWS_PT)])


def _bn_body(ah_ref, acc_ref, g_ref, b_ref, hin_ref, out_ref):
    ah = ah_ref[...]
    hp_lo = ah[:, 0:H] + acc_ref[0, 0:N, 0:H] / (acc_ref[0, 0:N, H:D] + 1e-6)
    hp_hi = ah[:, H:D] + acc_ref[1, 0:N, 0:H] / (acc_ref[1, 0:N, H:D] + 1e-6)
    hp = jnp.concatenate([hp_lo, hp_hi], axis=1)
    mean = jnp.mean(hp, axis=0, keepdims=True)
    var = jnp.mean((hp - mean) ** 2, axis=0, keepdims=True)
    y = (hp - mean) * (g_ref[...] * lax.rsqrt(var + 1e-5)) + b_ref[...]
    out_ref[...] = hin_ref[...] + jnp.maximum(y, 0.0)


@jax.jit
def kernel(h, edge_index, e, WA, bA, WB, bB, WD, bD, WE, bE, gamma, beta):
    f32 = jnp.float32
    ah, bdt, et = pl.pallas_call(
        _proj_body,
        out_shape=(
            jax.ShapeDtypeStruct((N, D), f32),
            jax.ShapeDtypeStruct((NC, N, D), f32),
            jax.ShapeDtypeStruct((N, D), f32),
        ),
    )(h, WA, bA.reshape(1, D), WB, bB.reshape(1, D),
      WD, bD.reshape(1, D), WE, bE.reshape(1, D))

    mesh = plsc.VectorSubcoreMesh(core_axis_name="c", subcore_axis_name="s")
    acc = pl.kernel(
        _edge_body,
        out_type=jax.ShapeDtypeStruct((NC, NP_, D), f32),
        mesh=mesh,
        scratch_types=[
            pltpu.VMEM((K,), jnp.int32),
            pltpu.VMEM((K,), jnp.int32),
            pltpu.VMEM((K,), jnp.int32),
            pltpu.VMEM((K,), jnp.int32),
            pltpu.VMEM((K,), jnp.int32),
            pltpu.VMEM((K,), jnp.int32),
            pltpu.VMEM((K, D), f32),
            pltpu.VMEM((K, D), f32),
            pltpu.VMEM((K, D), f32),
            pltpu.VMEM((K, D), f32),
            pltpu.VMEM((K, D), f32),
            pltpu.VMEM((K, D), f32),
            pltpu.VMEM_SHARED((NP_, D), f32),
            pltpu.SemaphoreType.DMA,
            pltpu.SemaphoreType.DMA,
            pltpu.SemaphoreType.DMA,
            pltpu.SemaphoreType.DMA,
            pltpu.SemaphoreType.DMA,
            pltpu.SemaphoreType.DMA,
            pltpu.SemaphoreType.DMA,
            pltpu.SemaphoreType.DMA,
        ],
    )(bdt.reshape(NC * N, D), et, edge_index[0], edge_index[1])

    h_new = pl.pallas_call(
        _bn_body,
        out_shape=jax.ShapeDtypeStruct((N, D), f32),
    )(ah, acc, gamma.reshape(1, D), beta.reshape(1, D), h)
    return (h_new, e)


# speedup vs baseline: 1.0610x; 1.0132x over previous
import functools

import jax
import jax.numpy as jnp
from jax import lax
from jax.experimental import pallas as pl
from jax.experimental.pallas import tpu as pltpu
from jax.experimental.pallas import tpu_sc as plsc

N = 10000
E = 320000
D = 128
H = D // 2

NC = 2
NS = 16
K = 32
EPW = E // NS
CHUNKS = EPW // K
QUADS = CHUNKS // 4
LAST = QUADS * 4
NP_ = 10240
ROWS_PT = NP_ // NS


def _proj_body(h_ref, wa, ba, wb, bb, wd, bd_, we, be, ah_ref, bdt_ref, et_ref):
    hb = h_ref[...]
    ah_ref[...] = jnp.dot(hb, wa[...], preferred_element_type=jnp.float32) + ba[...]
    yb = jnp.dot(hb, wb[...], preferred_element_type=jnp.float32) + bb[...]
    yd = jnp.dot(hb, wd[...], preferred_element_type=jnp.float32) + bd_[...]
    et_ref[...] = jnp.dot(hb, we[...], preferred_element_type=jnp.float32) + be[...]
    for c in range(NC):
        bdt_ref[c, :, 0:H] = yb[:, c * H:(c + 1) * H]
        bdt_ref[c, :, H:D] = yd[:, c * H:(c + 1) * H]


def _edge_body(bdt, et, src_hbm, dst_hbm, out,
               ibd0, ibd1, ibd2, ibd3, dst0, dst1, dst2, dst3, sd0, sd1,
               bd0, bd1, bd2, bd3, ev0, ev1, ev2, ev3, ms0, ms1,
               acc_sh, isem0, isem1, isem2, isem3,
               gsem0, gsem1, gsem2, gsem3, esem0, esem1, esem2, esem3,
               ssem0, ssem1):
    c = lax.axis_index("c")
    s = lax.axis_index("s")
    cN = c * N
    cH = c * H
    ibd = (ibd0, ibd1, ibd2, ibd3)
    dstg = (dst0, dst1, dst2, dst3)
    sdst = (sd0, sd1)
    bdv = (bd0, bd1, bd2, bd3)
    evv = (ev0, ev1, ev2, ev3)
    msv = (ms0, ms1)
    isem = (isem0, isem1, isem2, isem3)
    gsem = (gsem0, gsem1, gsem2, gsem3)
    esem = (esem0, esem1, esem2, esem3)
    ssem = (ssem0, ssem1)

    def issue_idx(base, slot):
        pltpu.async_copy(src_hbm.at[pl.ds(base, K)], ibd[slot], isem[slot])
        pltpu.async_copy(dst_hbm.at[pl.ds(base, K)], dstg[slot], isem[slot])

    def wait_idx(slot):
        pltpu.make_async_copy(src_hbm.at[pl.ds(0, K)], ibd[slot],
                              isem[slot]).wait()
        pltpu.make_async_copy(dst_hbm.at[pl.ds(0, K)], dstg[slot],
                              isem[slot]).wait()

    def issue_gather(slot):
        for f in range(K // 16):
            sl = pl.ds(f * 16, 16)
            ibd[slot][sl] = ibd[slot][sl] + cN
        pltpu.async_copy(bdt.at[ibd[slot]], bdv[slot], gsem[slot])
        pltpu.async_copy(et.at[dstg[slot]], evv[slot], esem[slot])

    def wait_gather(slot):
        pltpu.make_async_copy(bdt.at[ibd[slot]], bdv[slot], gsem[slot]).wait()
        pltpu.make_async_copy(et.at[dstg[slot]], evv[slot], esem[slot]).wait()

    def compute(slot, mslot):
        bd_v = bdv[slot]
        e_v = evv[slot]
        ms_v = msv[mslot]

        @plsc.parallel_loop(0, K, unroll=4)
        def edge(i):
            for f in range(H // 16):
                lo = pl.ds(f * 16, 16)
                hi = pl.ds(H + f * 16, 16)
                b = bd_v[i, lo]
                d = bd_v[i, hi]
                ev = e_v[i, pl.ds(cH + f * 16, 16)]
                sg = 1.0 / (1.0 + jnp.exp(-(d + ev)))
                ms_v[i, lo] = sg * b
                ms_v[i, hi] = sg
        for f in range(K // 16):
            sl = pl.ds(f * 16, 16)
            sdst[mslot][sl] = dstg[slot][sl]

    def issue_scatter(mslot):
        pltpu.async_copy(msv[mslot], acc_sh.at[sdst[mslot]], ssem[mslot],
                         add=True)

    def wait_scatter(mslot):
        pltpu.make_async_copy(msv[mslot], acc_sh.at[sdst[mslot]],
                              ssem[mslot]).wait()

    for j in range(4):
        issue_idx(s * EPW + j * K, j)

    def zrow(i, _):
        for f in range(D // 16):
            ms0[i, pl.ds(f * 16, 16)] = jnp.zeros((16,), jnp.float32)
        return 0
    lax.fori_loop(0, K, zrow, 0)
    for r in range(ROWS_PT // K):
        pltpu.sync_copy(ms0, acc_sh.at[pl.ds(s * ROWS_PT + r * K, K)])
    plsc.subcore_barrier()

    wait_idx(0)
    issue_gather(0)
    wait_idx(1)
    issue_gather(1)

    def quad(p, _):
        for j in range(4):
            t = 4 * p + j
            nslot = (j + 2) % 4
            mslot = j % 2
            wait_gather(j)

            @pl.when(t + 2 < LAST)
            def _():
                wait_idx(nslot)
                issue_gather(nslot)

            @pl.when(t >= 2)
            def _():
                wait_scatter(mslot)

            compute(j, mslot)
            issue_scatter(mslot)

            @pl.when(t + 4 <= LAST)
            def _():
                issue_idx(s * EPW + (t + 4) * K, j)
        return 0
    lax.fori_loop(0, QUADS, quad, 0)
    wait_scatter(0)
    wait_scatter(1)

    wait_idx(0)
    issue_gather(0)
    wait_gather(0)
    compute(0, 0)
    pltpu.sync_copy(msv[0], acc_sh.at[sdst[0]], add=True)

    plsc.subcore_barrier()
    pltpu.sync_copy(acc_sh.at[pl.ds(s * ROWS_PT, ROWS_PT)],
                    out.at[c, pl.ds(s * ROWS_PT, ROWS_PT)])


def _bn_body(ah_ref, acc_ref, g_ref, b_ref, hin_ref, out_ref):
    ah = ah_ref[...]
    hp_lo = ah[:, 0:H] + acc_ref[0, 0:N, 0:H] / (acc_ref[0, 0:N, H:D] + 1e-6)
    hp_hi = ah[:, H:D] + acc_ref[1, 0:N, 0:H] / (acc_ref[1, 0:N, H:D] + 1e-6)
    hp = jnp.concatenate([hp_lo, hp_hi], axis=1)
    mean = jnp.mean(hp, axis=0, keepdims=True)
    var = jnp.mean((hp - mean) ** 2, axis=0, keepdims=True)
    y = (hp - mean) * (g_ref[...] * lax.rsqrt(var + 1e-5)) + b_ref[...]
    out_ref[...] = hin_ref[...] + jnp.maximum(y, 0.0)


@jax.jit
def kernel(h, edge_index, e, WA, bA, WB, bB, WD, bD, WE, bE, gamma, beta):
    f32 = jnp.float32
    ah, bdt, et = pl.pallas_call(
        _proj_body,
        out_shape=(
            jax.ShapeDtypeStruct((N, D), f32),
            jax.ShapeDtypeStruct((NC, N, D), f32),
            jax.ShapeDtypeStruct((N, D), f32),
        ),
    )(h, WA, bA.reshape(1, D), WB, bB.reshape(1, D),
      WD, bD.reshape(1, D), WE, bE.reshape(1, D))

    mesh = plsc.VectorSubcoreMesh(core_axis_name="c", subcore_axis_name="s")
    acc = pl.kernel(
        _edge_body,
        out_type=jax.ShapeDtypeStruct((NC, NP_, D), f32),
        mesh=mesh,
        scratch_types=[
            pltpu.VMEM((K,), jnp.int32),
            pltpu.VMEM((K,), jnp.int32),
            pltpu.VMEM((K,), jnp.int32),
            pltpu.VMEM((K,), jnp.int32),
            pltpu.VMEM((K,), jnp.int32),
            pltpu.VMEM((K,), jnp.int32),
            pltpu.VMEM((K,), jnp.int32),
            pltpu.VMEM((K,), jnp.int32),
            pltpu.VMEM((K,), jnp.int32),
            pltpu.VMEM((K,), jnp.int32),
            pltpu.VMEM((K, D), f32),
            pltpu.VMEM((K, D), f32),
            pltpu.VMEM((K, D), f32),
            pltpu.VMEM((K, D), f32),
            pltpu.VMEM((K, D), f32),
            pltpu.VMEM((K, D), f32),
            pltpu.VMEM((K, D), f32),
            pltpu.VMEM((K, D), f32),
            pltpu.VMEM((K, D), f32),
            pltpu.VMEM((K, D), f32),
            pltpu.VMEM_SHARED((NP_, D), f32),
        ] + [pltpu.SemaphoreType.DMA] * 14,
    )(bdt.reshape(NC * N, D), et, edge_index[0], edge_index[1])

    h_new = pl.pallas_call(
        _bn_body,
        out_shape=jax.ShapeDtypeStruct((N, D), f32),
    )(ah, acc, gamma.reshape(1, D), beta.reshape(1, D), h)
    return (h_new, e)
